# baseline (speedup 1.0000x reference)
import functools

import jax
import jax.numpy as jnp
from jax import lax
from jax.experimental import pallas as pl
from jax.experimental.pallas import tpu as pltpu
from jax.experimental.pallas import tpu_sc as plsc

N_NODES = 10000
D_IN = 128
H1, H2, OUT = 10, 5, 2
N_EDGES = 320000

F = 16
PAD_N = 10112
NFOLD = PAD_N // 8
NC, NS = 2, 16
NW = NC * NS
E_PER_W = N_EDGES // NW
CHUNK = 40
N_CHUNKS = E_PER_W // CHUNK
GK = 50
N_GROUPS = N_CHUNKS // GK
PACK_PER_S = N_NODES // NS
ACC_N = PAD_N
ZSTRIPE = ACC_N // NS
ROWS_PER_S = PAD_N // NS

_MESH = plsc.VectorSubcoreMesh(
    core_axis_name="c", subcore_axis_name="s", num_cores=NC, num_subcores=NS
)
_SC_PARAMS = pltpu.CompilerParams(use_tc_tiling_on_sc=False)


def _zero_acc(zeros_hbm, zb_v, acc_s, s):
    pltpu.sync_copy(zeros_hbm, zb_v)
    base = s * ZSTRIPE
    for k in range(ZSTRIPE // 128):
        pltpu.sync_copy(zb_v, acc_s.at[pl.ds(base + k * 128, 128)])
    rem = ZSTRIPE % 128
    if rem:
        pltpu.sync_copy(
            zb_v.at[pl.ds(0, rem)],
            acc_s.at[pl.ds(base + (ZSTRIPE // 128) * 128, rem)],
        )


@functools.partial(
    pl.kernel,
    out_type=jax.ShapeDtypeStruct((NC, PAD_N, F), jnp.float32),
    mesh=_MESH,
    scratch_types=[
        pltpu.VMEM_SHARED((ACC_N, F), jnp.float32),
        pltpu.VMEM((N_CHUNKS, CHUNK), jnp.int32),
        pltpu.VMEM((CHUNK, F), jnp.float32),
        pltpu.VMEM((128, F), jnp.float32),
        pltpu.SemaphoreType.DMA,
    ],
    compiler_params=_SC_PARAMS,
)
def _deg_call(e_hbm, ones_hbm, zeros_hbm, out_hbm, acc_s, dst_v, ones_v, zb_v, sem):
    c = lax.axis_index("c")
    s = lax.axis_index("s")
    wid = s * NC + c
    _zero_acc(zeros_hbm, zb_v, acc_s, s)
    pltpu.sync_copy(ones_hbm, ones_v)
    pltpu.sync_copy(e_hbm.at[1, wid], dst_v)
    plsc.subcore_barrier()

    descs = [
        pltpu.async_copy(ones_v, acc_s.at[dst_v.at[j]], sem, add=True)
        for j in range(N_CHUNKS)
    ]
    for d in descs:
        d.wait()
    plsc.subcore_barrier()
    base = s * ROWS_PER_S
    pltpu.sync_copy(
        acc_s.at[pl.ds(base, ROWS_PER_S)], out_hbm.at[c, pl.ds(base, ROWS_PER_S)]
    )


@functools.partial(
    pl.kernel,
    out_type=jax.ShapeDtypeStruct((NC, PAD_N, F), jnp.float32),
    mesh=_MESH,
    scratch_types=[
        pltpu.VMEM_SHARED((ACC_N, F), jnp.float32),
        pltpu.VMEM_SHARED((PAD_N, F), jnp.float32),
        pltpu.VMEM((N_CHUNKS, CHUNK), jnp.int32),
        pltpu.VMEM((N_CHUNKS, CHUNK), jnp.int32),
        pltpu.VMEM((2, GK * CHUNK, F), jnp.float32),
        pltpu.VMEM((128, F), jnp.float32),
        pltpu.VMEM((ROWS_PER_S, F), jnp.float32),
        pltpu.SemaphoreType.DMA,
        pltpu.SemaphoreType.DMA,
    ],
    compiler_params=_SC_PARAMS,
)
def _layer_call(
    u_hbm, e_hbm, zeros_hbm, out_hbm,
    acc_s, u_s, src_v, dst_v, rows_v, zb_v, stage_v, gsem, ssem,
):
    c = lax.axis_index("c")
    s = lax.axis_index("s")
    wid = s * NC + c
    base0 = s * ROWS_PER_S
    pltpu.sync_copy(u_hbm.at[pl.ds(base0, ROWS_PER_S)], stage_v)
    pltpu.sync_copy(stage_v, u_s.at[pl.ds(base0, ROWS_PER_S)])
    _zero_acc(zeros_hbm, zb_v, acc_s, s)
    pltpu.sync_copy(e_hbm.at[0, wid], src_v)
    pltpu.sync_copy(e_hbm.at[1, wid], dst_v)
    plsc.subcore_barrier()

    def fire_gathers(g, p):
        return [
            pltpu.async_copy(
                u_s.at[src_v.at[g * GK + k]],
                rows_v.at[p, pl.ds(k * CHUNK, CHUNK)],
                gsem,
            )
            for k in range(GK)
        ]

    dg = {0: fire_gathers(0, 0)}
    ds = {}
    for g in range(N_GROUPS):
        p = g % 2
        for d in dg[g]:
            d.wait()
        ds[g] = [
            pltpu.async_copy(
                rows_v.at[p, pl.ds(k * CHUNK, CHUNK)],
                acc_s.at[dst_v.at[g * GK + k]],
                ssem,
                add=True,
            )
            for k in range(GK)
        ]
        if g + 1 < N_GROUPS:
            if g >= 1:
                for d in ds[g - 1]:
                    d.wait()
            dg[g + 1] = fire_gathers(g + 1, 1 - p)
    for g in (N_GROUPS - 2, N_GROUPS - 1):
        for d in ds[g]:
            d.wait()
    plsc.subcore_barrier()
    base = s * ROWS_PER_S
    pltpu.sync_copy(
        acc_s.at[pl.ds(base, ROWS_PER_S)], out_hbm.at[c, pl.ds(base, ROWS_PER_S)]
    )


@functools.partial(
    pl.kernel,
    out_type=(
        jax.ShapeDtypeStruct((N_NODES, OUT), jnp.float32),
        jax.ShapeDtypeStruct((N_NODES, OUT), jnp.float32),
    ),
    mesh=_MESH,
    scratch_types=[pltpu.VMEM((PACK_PER_S, OUT), jnp.float32)],
    compiler_params=_SC_PARAMS,
)
def _pack_call(h_hbm, o_hbm, hout_hbm, oout_hbm, buf_v):
    c = lax.axis_index("c")
    s = lax.axis_index("s")
    base = s * PACK_PER_S

    @pl.when(c == 0)
    def _():
        pltpu.sync_copy(h_hbm.at[pl.ds(base, PACK_PER_S), pl.ds(0, OUT)], buf_v)
        pltpu.sync_copy(buf_v, hout_hbm.at[pl.ds(base, PACK_PER_S)])

    @pl.when(c == 1)
    def _():
        pltpu.sync_copy(o_hbm.at[pl.ds(base, PACK_PER_S), pl.ds(0, OUT)], buf_v)
        pltpu.sync_copy(buf_v, oout_hbm.at[pl.ds(base, PACK_PER_S)])


def _tc_first_body(x_ref, w_ref, degp_ref, u_ref, dinv_ref):
    dinv = lax.rsqrt(degp_ref[0] + degp_ref[1] + 1.0)
    hwf = jnp.dot(x_ref[...], w_ref[...], preferred_element_type=jnp.float32)
    dinv_ref[...] = dinv
    u_ref[...] = hwf * dinv


def _tc_mid_body(part_ref, u_ref, dinv_ref, b_ref, w_ref, unext_ref):
    agg = dinv_ref[...] * (part_ref[0] + part_ref[1] + u_ref[...]) + b_ref[...]
    xn = jnp.tanh(agg)
    unext_ref[...] = (
        jnp.dot(xn, w_ref[...], preferred_element_type=jnp.float32) * dinv_ref[...]
    )


def _tc_final_body(part_ref, u_ref, dinv_ref, b_ref, wc_ref, bc_ref, h_ref, out_ref):
    agg = dinv_ref[...] * (part_ref[0] + part_ref[1] + u_ref[...]) + b_ref[...]
    xn = jnp.tanh(agg)
    h_ref[...] = xn
    out_ref[...] = (
        jnp.dot(xn, wc_ref[...], preferred_element_type=jnp.float32) + bc_ref[...]
    )


_FOLDED = jax.ShapeDtypeStruct((NFOLD, 128), jnp.float32)

_tc_first = pl.pallas_call(_tc_first_body, out_shape=(_FOLDED, _FOLDED))
_tc_mid = pl.pallas_call(_tc_mid_body, out_shape=_FOLDED)
_tc_final = pl.pallas_call(_tc_final_body, out_shape=(_FOLDED, _FOLDED))


def _pad2(w, rows, cols):
    return jnp.pad(w, ((0, rows - w.shape[0]), (0, cols - w.shape[1])))


def _blockdiag(w):
    return jnp.kron(jnp.eye(8, dtype=jnp.float32), w)


def _btile(b, width):
    return jnp.tile(jnp.pad(b, (0, F - width)), 8).reshape(1, 128)


def kernel(x, edge_index, W1, b1, W2, b2, W3, b3, Wc, bc):
    eidx = jnp.reshape(edge_index.astype(jnp.int32), (2, NW, N_CHUNKS, CHUNK))

    xf = jnp.reshape(jnp.pad(x, ((0, PAD_N - N_NODES), (0, 0))), (NFOLD, 8 * D_IN))
    W1b = jnp.kron(jnp.eye(8, dtype=jnp.float32), _pad2(W1, D_IN, F))
    W2b = _blockdiag(_pad2(W2, F, F))
    W3b = _blockdiag(_pad2(W3, F, F))
    Wcb = _blockdiag(_pad2(Wc, F, F))
    b1t = _btile(b1, H1)
    b2t = _btile(b2, H2)
    b3t = _btile(b3, OUT)
    bct = _btile(bc, OUT)

    ones = jnp.ones((CHUNK, F), jnp.float32)
    zeros = jnp.zeros((128, F), jnp.float32)

    degp = jnp.reshape(_deg_call(eidx, ones, zeros), (NC, NFOLD, 128))
    u1, dinv = _tc_first(xf, W1b, degp)
    p1 = jnp.reshape(_layer_call(jnp.reshape(u1, (PAD_N, F)), eidx, zeros),
                     (NC, NFOLD, 128))
    u2 = _tc_mid(p1, u1, dinv, b1t, W2b)
    p2 = jnp.reshape(_layer_call(jnp.reshape(u2, (PAD_N, F)), eidx, zeros),
                     (NC, NFOLD, 128))
    u3 = _tc_mid(p2, u2, dinv, b2t, W3b)
    p3 = jnp.reshape(_layer_call(jnp.reshape(u3, (PAD_N, F)), eidx, zeros),
                     (NC, NFOLD, 128))
    h_f, out_f = _tc_final(p3, u3, dinv, b3t, Wcb, bct)

    h_v = jnp.reshape(h_f, (PAD_N, F))
    o_v = jnp.reshape(out_f, (PAD_N, F))
    h_out, o_out = _pack_call(h_v, o_v)
    return (o_out, h_out)


# speedup vs baseline: 1.0582x; 1.0582x over previous
import functools

import jax
import jax.numpy as jnp
from jax import lax
from jax.experimental import pallas as pl
from jax.experimental.pallas import tpu as pltpu
from jax.experimental.pallas import tpu_sc as plsc

N_NODES = 10000
D_IN = 128
H1, H2, OUT = 10, 5, 2
N_EDGES = 320000

F = 16
PAD_N = 10112
NFOLD = PAD_N // 8
NC, NS = 2, 16
NW = NC * NS
E_PER_W = N_EDGES // NW
CHUNK = 80
N_CHUNKS = E_PER_W // CHUNK
GK = 25
N_GROUPS = N_CHUNKS // GK
PACK_PER_S = N_NODES // NS
ACC_N = PAD_N
ZSTRIPE = ACC_N // NS
ROWS_PER_S = PAD_N // NS

_MESH = plsc.VectorSubcoreMesh(
    core_axis_name="c", subcore_axis_name="s", num_cores=NC, num_subcores=NS
)
_SC_PARAMS = pltpu.CompilerParams(use_tc_tiling_on_sc=False)


def _zero_acc(zeros_hbm, zb_v, acc_s, s):
    pltpu.sync_copy(zeros_hbm, zb_v)
    base = s * ZSTRIPE
    for k in range(ZSTRIPE // 128):
        pltpu.sync_copy(zb_v, acc_s.at[pl.ds(base + k * 128, 128)])
    rem = ZSTRIPE % 128
    if rem:
        pltpu.sync_copy(
            zb_v.at[pl.ds(0, rem)],
            acc_s.at[pl.ds(base + (ZSTRIPE // 128) * 128, rem)],
        )


@functools.partial(
    pl.kernel,
    out_type=jax.ShapeDtypeStruct((NC, PAD_N, F), jnp.float32),
    mesh=_MESH,
    scratch_types=[
        pltpu.VMEM_SHARED((ACC_N, F), jnp.float32),
        pltpu.VMEM((N_CHUNKS, CHUNK), jnp.int32),
        pltpu.VMEM((CHUNK, F), jnp.float32),
        pltpu.VMEM((128, F), jnp.float32),
        pltpu.SemaphoreType.DMA,
    ],
    compiler_params=_SC_PARAMS,
)
def _deg_call(e_hbm, ones_hbm, zeros_hbm, out_hbm, acc_s, dst_v, ones_v, zb_v, sem):
    c = lax.axis_index("c")
    s = lax.axis_index("s")
    wid = s * NC + c
    _zero_acc(zeros_hbm, zb_v, acc_s, s)
    pltpu.sync_copy(ones_hbm, ones_v)
    pltpu.sync_copy(e_hbm.at[1, wid], dst_v)
    plsc.subcore_barrier()

    descs = [
        pltpu.async_copy(ones_v, acc_s.at[dst_v.at[j]], sem, add=True)
        for j in range(N_CHUNKS)
    ]
    for d in descs:
        d.wait()
    plsc.subcore_barrier()
    base = s * ROWS_PER_S
    pltpu.sync_copy(
        acc_s.at[pl.ds(base, ROWS_PER_S)], out_hbm.at[c, pl.ds(base, ROWS_PER_S)]
    )


@functools.partial(
    pl.kernel,
    out_type=jax.ShapeDtypeStruct((NC, PAD_N, F), jnp.float32),
    mesh=_MESH,
    scratch_types=[
        pltpu.VMEM_SHARED((ACC_N, F), jnp.float32),
        pltpu.VMEM_SHARED((PAD_N, F), jnp.float32),
        pltpu.VMEM((N_CHUNKS, CHUNK), jnp.int32),
        pltpu.VMEM((N_CHUNKS, CHUNK), jnp.int32),
        pltpu.VMEM((2, GK * CHUNK, F), jnp.float32),
        pltpu.VMEM((128, F), jnp.float32),
        pltpu.VMEM((ROWS_PER_S, F), jnp.float32),
        pltpu.SemaphoreType.DMA,
        pltpu.SemaphoreType.DMA,
    ],
    compiler_params=_SC_PARAMS,
)
def _layer_call(
    u_hbm, e_hbm, zeros_hbm, out_hbm,
    acc_s, u_s, src_v, dst_v, rows_v, zb_v, stage_v, gsem, ssem,
):
    c = lax.axis_index("c")
    s = lax.axis_index("s")
    wid = s * NC + c
    base0 = s * ROWS_PER_S
    pltpu.sync_copy(u_hbm.at[pl.ds(base0, ROWS_PER_S)], stage_v)
    pltpu.sync_copy(stage_v, u_s.at[pl.ds(base0, ROWS_PER_S)])
    _zero_acc(zeros_hbm, zb_v, acc_s, s)
    pltpu.sync_copy(e_hbm.at[0, wid], src_v)
    pltpu.sync_copy(e_hbm.at[1, wid], dst_v)
    plsc.subcore_barrier()

    def fire_gathers(g, p):
        return [
            pltpu.async_copy(
                u_s.at[src_v.at[g * GK + k]],
                rows_v.at[p, pl.ds(k * CHUNK, CHUNK)],
                gsem,
            )
            for k in range(GK)
        ]

    dg = {0: fire_gathers(0, 0)}
    ds = {}
    for g in range(N_GROUPS):
        p = g % 2
        for d in dg[g]:
            d.wait()
        ds[g] = [
            pltpu.async_copy(
                rows_v.at[p, pl.ds(k * CHUNK, CHUNK)],
                acc_s.at[dst_v.at[g * GK + k]],
                ssem,
                add=True,
            )
            for k in range(GK)
        ]
        if g + 1 < N_GROUPS:
            if g >= 1:
                for d in ds[g - 1]:
                    d.wait()
            dg[g + 1] = fire_gathers(g + 1, 1 - p)
    for g in (N_GROUPS - 2, N_GROUPS - 1):
        for d in ds[g]:
            d.wait()
    plsc.subcore_barrier()
    base = s * ROWS_PER_S
    pltpu.sync_copy(
        acc_s.at[pl.ds(base, ROWS_PER_S)], out_hbm.at[c, pl.ds(base, ROWS_PER_S)]
    )


@functools.partial(
    pl.kernel,
    out_type=(
        jax.ShapeDtypeStruct((N_NODES, OUT), jnp.float32),
        jax.ShapeDtypeStruct((N_NODES, OUT), jnp.float32),
    ),
    mesh=_MESH,
    scratch_types=[pltpu.VMEM((PACK_PER_S, OUT), jnp.float32)],
    compiler_params=_SC_PARAMS,
)
def _pack_call(h_hbm, o_hbm, hout_hbm, oout_hbm, buf_v):
    c = lax.axis_index("c")
    s = lax.axis_index("s")
    base = s * PACK_PER_S

    @pl.when(c == 0)
    def _():
        pltpu.sync_copy(h_hbm.at[pl.ds(base, PACK_PER_S), pl.ds(0, OUT)], buf_v)
        pltpu.sync_copy(buf_v, hout_hbm.at[pl.ds(base, PACK_PER_S)])

    @pl.when(c == 1)
    def _():
        pltpu.sync_copy(o_hbm.at[pl.ds(base, PACK_PER_S), pl.ds(0, OUT)], buf_v)
        pltpu.sync_copy(buf_v, oout_hbm.at[pl.ds(base, PACK_PER_S)])


def _tc_mm_body(x_ref, w_ref, hw_ref):
    hw_ref[...] = jnp.dot(x_ref[...], w_ref[...], preferred_element_type=jnp.float32)


def _tc_comb_body(hw_ref, degp_ref, u_ref, dinv_ref):
    dinv = lax.rsqrt(degp_ref[0] + degp_ref[1] + 1.0)
    dinv_ref[...] = dinv
    u_ref[...] = hw_ref[...] * dinv


def _tc_mid_body(part_ref, u_ref, dinv_ref, b_ref, w_ref, unext_ref):
    agg = dinv_ref[...] * (part_ref[0] + part_ref[1] + u_ref[...]) + b_ref[...]
    xn = jnp.tanh(agg)
    unext_ref[...] = (
        jnp.dot(xn, w_ref[...], preferred_element_type=jnp.float32) * dinv_ref[...]
    )


def _tc_final_body(part_ref, u_ref, dinv_ref, b_ref, wc_ref, bc_ref, h_ref, out_ref):
    agg = dinv_ref[...] * (part_ref[0] + part_ref[1] + u_ref[...]) + b_ref[...]
    xn = jnp.tanh(agg)
    h_ref[...] = xn
    out_ref[...] = (
        jnp.dot(xn, wc_ref[...], preferred_element_type=jnp.float32) + bc_ref[...]
    )


_FOLDED = jax.ShapeDtypeStruct((NFOLD, 128), jnp.float32)

_tc_mm = pl.pallas_call(_tc_mm_body, out_shape=_FOLDED)
_tc_comb = pl.pallas_call(_tc_comb_body, out_shape=(_FOLDED, _FOLDED))
_tc_mid = pl.pallas_call(_tc_mid_body, out_shape=_FOLDED)
_tc_final = pl.pallas_call(_tc_final_body, out_shape=(_FOLDED, _FOLDED))


def _pad2(w, rows, cols):
    return jnp.pad(w, ((0, rows - w.shape[0]), (0, cols - w.shape[1])))


def _blockdiag(w):
    return jnp.kron(jnp.eye(8, dtype=jnp.float32), w)


def _btile(b, width):
    return jnp.tile(jnp.pad(b, (0, F - width)), 8).reshape(1, 128)


def kernel(x, edge_index, W1, b1, W2, b2, W3, b3, Wc, bc):
    eidx = jnp.reshape(edge_index.astype(jnp.int32), (2, NW, N_CHUNKS, CHUNK))

    xf = jnp.reshape(jnp.pad(x, ((0, PAD_N - N_NODES), (0, 0))), (NFOLD, 8 * D_IN))
    W1b = jnp.kron(jnp.eye(8, dtype=jnp.float32), _pad2(W1, D_IN, F))
    W2b = _blockdiag(_pad2(W2, F, F))
    W3b = _blockdiag(_pad2(W3, F, F))
    Wcb = _blockdiag(_pad2(Wc, F, F))
    b1t = _btile(b1, H1)
    b2t = _btile(b2, H2)
    b3t = _btile(b3, OUT)
    bct = _btile(bc, OUT)

    ones = jnp.ones((CHUNK, F), jnp.float32)
    zeros = jnp.zeros((128, F), jnp.float32)

    hw1 = _tc_mm(xf, W1b)
    degp = jnp.reshape(_deg_call(eidx, ones, zeros), (NC, NFOLD, 128))
    u1, dinv = _tc_comb(hw1, degp)
    p1 = jnp.reshape(_layer_call(jnp.reshape(u1, (PAD_N, F)), eidx, zeros),
                     (NC, NFOLD, 128))
    u2 = _tc_mid(p1, u1, dinv, b1t, W2b)
    p2 = jnp.reshape(_layer_call(jnp.reshape(u2, (PAD_N, F)), eidx, zeros),
                     (NC, NFOLD, 128))
    u3 = _tc_mid(p2, u2, dinv, b2t, W3b)
    p3 = jnp.reshape(_layer_call(jnp.reshape(u3, (PAD_N, F)), eidx, zeros),
                     (NC, NFOLD, 128))
    h_f, out_f = _tc_final(p3, u3, dinv, b3t, Wcb, bct)

    h_v = jnp.reshape(h_f, (PAD_N, F))
    o_v = jnp.reshape(out_f, (PAD_N, F))
    h_out, o_out = _pack_call(h_v, o_v)
    return (o_out, h_out)


# speedup vs baseline: 1.0583x; 1.0001x over previous
import functools

import jax
import jax.numpy as jnp
from jax import lax
from jax.experimental import pallas as pl
from jax.experimental.pallas import tpu as pltpu
from jax.experimental.pallas import tpu_sc as plsc

N_NODES = 10000
D_IN = 128
H1, H2, OUT = 10, 5, 2
N_EDGES = 320000

F = 16
PAD_N = 10112
NFOLD = PAD_N // 8
NC, NS = 2, 16
NW = NC * NS
E_PER_W = N_EDGES // NW
CHUNK = 80
N_CHUNKS = E_PER_W // CHUNK
PIPE = 25
NSLOT = 2 * PIPE
PACK_PER_S = N_NODES // NS
ACC_N = PAD_N
ZSTRIPE = ACC_N // NS
ROWS_PER_S = PAD_N // NS

_MESH = plsc.VectorSubcoreMesh(
    core_axis_name="c", subcore_axis_name="s", num_cores=NC, num_subcores=NS
)
_SC_PARAMS = pltpu.CompilerParams(use_tc_tiling_on_sc=False)


def _zero_acc(zeros_hbm, zb_v, acc_s, s):
    pltpu.sync_copy(zeros_hbm, zb_v)
    base = s * ZSTRIPE
    for k in range(ZSTRIPE // 128):
        pltpu.sync_copy(zb_v, acc_s.at[pl.ds(base + k * 128, 128)])
    rem = ZSTRIPE % 128
    if rem:
        pltpu.sync_copy(
            zb_v.at[pl.ds(0, rem)],
            acc_s.at[pl.ds(base + (ZSTRIPE // 128) * 128, rem)],
        )


@functools.partial(
    pl.kernel,
    out_type=jax.ShapeDtypeStruct((NC, PAD_N, F), jnp.float32),
    mesh=_MESH,
    scratch_types=[
        pltpu.VMEM_SHARED((ACC_N, F), jnp.float32),
        pltpu.VMEM((N_CHUNKS, CHUNK), jnp.int32),
        pltpu.VMEM((CHUNK, F), jnp.float32),
        pltpu.VMEM((128, F), jnp.float32),
        pltpu.SemaphoreType.DMA,
    ],
    compiler_params=_SC_PARAMS,
)
def _deg_call(e_hbm, ones_hbm, zeros_hbm, out_hbm, acc_s, dst_v, ones_v, zb_v, sem):
    c = lax.axis_index("c")
    s = lax.axis_index("s")
    wid = s * NC + c
    _zero_acc(zeros_hbm, zb_v, acc_s, s)
    pltpu.sync_copy(ones_hbm, ones_v)
    pltpu.sync_copy(e_hbm.at[1, wid], dst_v)
    plsc.subcore_barrier()

    descs = [
        pltpu.async_copy(ones_v, acc_s.at[dst_v.at[j]], sem, add=True)
        for j in range(N_CHUNKS)
    ]
    for d in descs:
        d.wait()
    plsc.subcore_barrier()
    base = s * ROWS_PER_S
    pltpu.sync_copy(
        acc_s.at[pl.ds(base, ROWS_PER_S)], out_hbm.at[c, pl.ds(base, ROWS_PER_S)]
    )


@functools.partial(
    pl.kernel,
    out_type=jax.ShapeDtypeStruct((NC, PAD_N, F), jnp.float32),
    mesh=_MESH,
    scratch_types=[
        pltpu.VMEM_SHARED((ACC_N, F), jnp.float32),
        pltpu.VMEM_SHARED((PAD_N, F), jnp.float32),
        pltpu.VMEM((N_CHUNKS, CHUNK), jnp.int32),
        pltpu.VMEM((N_CHUNKS, CHUNK), jnp.int32),
        pltpu.VMEM((NSLOT, CHUNK, F), jnp.float32),
        pltpu.VMEM((128, F), jnp.float32),
        pltpu.VMEM((ROWS_PER_S, F), jnp.float32),
        pltpu.SemaphoreType.DMA,
        pltpu.SemaphoreType.DMA,
        pltpu.SemaphoreType.DMA,
    ],
    compiler_params=_SC_PARAMS,
)
def _layer_call(
    u_hbm, e_hbm, zeros_hbm, out_hbm,
    acc_s, u_s, src_v, dst_v, rows_v, zb_v, stage_v, gsem, ssem, psem,
):
    c = lax.axis_index("c")
    s = lax.axis_index("s")
    wid = s * NC + c
    base0 = s * ROWS_PER_S
    pltpu.sync_copy(u_hbm.at[pl.ds(base0, ROWS_PER_S)], stage_v)
    pltpu.sync_copy(stage_v, u_s.at[pl.ds(base0, ROWS_PER_S)])
    _zero_acc(zeros_hbm, zb_v, acc_s, s)
    pltpu.sync_copy(e_hbm.at[0, wid], src_v)
    pltpu.sync_copy(e_hbm.at[1, wid], dst_v)
    plsc.subcore_barrier()

    def fire_gathers(g, p):
        return [
            pltpu.async_copy(
                u_s.at[src_v.at[g * PIPE + k]],
                rows_v.at[p * PIPE + k],
                gsem,
            )
            for k in range(PIPE)
        ]

    n_groups = N_CHUNKS // PIPE
    dg = {0: fire_gathers(0, 0)}
    ds = {}
    for g in range(n_groups):
        p = g % 2
        for d in dg[g]:
            d.wait()
        ds[g] = [
            pltpu.async_copy(
                rows_v.at[p * PIPE + k],
                acc_s.at[dst_v.at[g * PIPE + k]],
                ssem,
                add=True,
            )
            for k in range(PIPE)
        ]
        if g + 1 < n_groups:
            if g >= 1:
                for d in ds[g - 1]:
                    d.wait()
            dg[g + 1] = fire_gathers(g + 1, 1 - p)
    for g in (n_groups - 2, n_groups - 1):
        for d in ds[g]:
            d.wait()
    plsc.subcore_barrier()
    base = s * ROWS_PER_S
    pltpu.sync_copy(
        acc_s.at[pl.ds(base, ROWS_PER_S)], out_hbm.at[c, pl.ds(base, ROWS_PER_S)]
    )


@functools.partial(
    pl.kernel,
    out_type=(
        jax.ShapeDtypeStruct((N_NODES, OUT), jnp.float32),
        jax.ShapeDtypeStruct((N_NODES, OUT), jnp.float32),
    ),
    mesh=_MESH,
    scratch_types=[pltpu.VMEM((PACK_PER_S, OUT), jnp.float32)],
    compiler_params=_SC_PARAMS,
)
def _pack_call(h_hbm, o_hbm, hout_hbm, oout_hbm, buf_v):
    c = lax.axis_index("c")
    s = lax.axis_index("s")
    base = s * PACK_PER_S

    @pl.when(c == 0)
    def _():
        pltpu.sync_copy(h_hbm.at[pl.ds(base, PACK_PER_S), pl.ds(0, OUT)], buf_v)
        pltpu.sync_copy(buf_v, hout_hbm.at[pl.ds(base, PACK_PER_S)])

    @pl.when(c == 1)
    def _():
        pltpu.sync_copy(o_hbm.at[pl.ds(base, PACK_PER_S), pl.ds(0, OUT)], buf_v)
        pltpu.sync_copy(buf_v, oout_hbm.at[pl.ds(base, PACK_PER_S)])


def _tc_mm_body(x_ref, w_ref, hw_ref):
    hw_ref[...] = jnp.dot(x_ref[...], w_ref[...], preferred_element_type=jnp.float32)


def _tc_comb_body(hw_ref, degp_ref, u_ref, dinv_ref):
    dinv = lax.rsqrt(degp_ref[0] + degp_ref[1] + 1.0)
    dinv_ref[...] = dinv
    u_ref[...] = hw_ref[...] * dinv


def _tc_mid_body(part_ref, u_ref, dinv_ref, b_ref, w_ref, unext_ref):
    agg = dinv_ref[...] * (part_ref[0] + part_ref[1] + u_ref[...]) + b_ref[...]
    xn = jnp.tanh(agg)
    unext_ref[...] = (
        jnp.dot(xn, w_ref[...], preferred_element_type=jnp.float32) * dinv_ref[...]
    )


def _tc_final_body(part_ref, u_ref, dinv_ref, b_ref, wc_ref, bc_ref, h_ref, out_ref):
    agg = dinv_ref[...] * (part_ref[0] + part_ref[1] + u_ref[...]) + b_ref[...]
    xn = jnp.tanh(agg)
    h_ref[...] = xn
    out_ref[...] = (
        jnp.dot(xn, wc_ref[...], preferred_element_type=jnp.float32) + bc_ref[...]
    )


_FOLDED = jax.ShapeDtypeStruct((NFOLD, 128), jnp.float32)

_tc_mm = pl.pallas_call(_tc_mm_body, out_shape=_FOLDED)
_tc_comb = pl.pallas_call(_tc_comb_body, out_shape=(_FOLDED, _FOLDED))
_tc_mid = pl.pallas_call(_tc_mid_body, out_shape=_FOLDED)
_tc_final = pl.pallas_call(_tc_final_body, out_shape=(_FOLDED, _FOLDED))


def _pad2(w, rows, cols):
    return jnp.pad(w, ((0, rows - w.shape[0]), (0, cols - w.shape[1])))


def _blockdiag(w):
    return jnp.kron(jnp.eye(8, dtype=jnp.float32), w)


def _btile(b, width):
    return jnp.tile(jnp.pad(b, (0, F - width)), 8).reshape(1, 128)


def kernel(x, edge_index, W1, b1, W2, b2, W3, b3, Wc, bc):
    eidx = jnp.reshape(edge_index.astype(jnp.int32), (2, NW, N_CHUNKS, CHUNK))

    xf = jnp.reshape(jnp.pad(x, ((0, PAD_N - N_NODES), (0, 0))), (NFOLD, 8 * D_IN))
    W1b = jnp.kron(jnp.eye(8, dtype=jnp.float32), _pad2(W1, D_IN, F))
    W2b = _blockdiag(_pad2(W2, F, F))
    W3b = _blockdiag(_pad2(W3, F, F))
    Wcb = _blockdiag(_pad2(Wc, F, F))
    b1t = _btile(b1, H1)
    b2t = _btile(b2, H2)
    b3t = _btile(b3, OUT)
    bct = _btile(bc, OUT)

    ones = jnp.ones((CHUNK, F), jnp.float32)
    zeros = jnp.zeros((128, F), jnp.float32)

    hw1 = _tc_mm(xf, W1b)
    degp = jnp.reshape(_deg_call(eidx, ones, zeros), (NC, NFOLD, 128))
    u1, dinv = _tc_comb(hw1, degp)
    p1 = jnp.reshape(_layer_call(jnp.reshape(u1, (PAD_N, F)), eidx, zeros),
                     (NC, NFOLD, 128))
    u2 = _tc_mid(p1, u1, dinv, b1t, W2b)
    p2 = jnp.reshape(_layer_call(jnp.reshape(u2, (PAD_N, F)), eidx, zeros),
                     (NC, NFOLD, 128))
    u3 = _tc_mid(p2, u2, dinv, b2t, W3b)
    p3 = jnp.reshape(_layer_call(jnp.reshape(u3, (PAD_N, F)), eidx, zeros),
                     (NC, NFOLD, 128))
    h_f, out_f = _tc_final(p3, u3, dinv, b3t, Wcb, bct)

    h_v = jnp.reshape(h_f, (PAD_N, F))
    o_v = jnp.reshape(out_f, (PAD_N, F))
    h_out, o_out = _pack_call(h_v, o_v)
    return (o_out, h_out)
